# baseline (speedup 1.0000x reference)
import functools

import jax
import jax.numpy as jnp
from jax import lax
from jax.experimental import pallas as pl
from jax.experimental.pallas import tpu as pltpu
from jax.experimental.pallas import tpu_sc as plsc

VOCAB = 100000
HIDDEN = 768
MAX_POS = 2048
BATCH = 4
SEQ = 2048
EPS = 1e-12

NC = 2
NS = 16
NW = NC * NS
PW = SEQ // NW
NV = HIDDEN // 16


def _body(word_hbm, ids_hbm, ttf_hbm, pos_hbm, type_hbm, lnw_hbm, lnb_hbm,
          out_hbm, idx_v, ttf_v, posv, rows, typ_v, lnw_v, lnb_v, sem):
    c = lax.axis_index("c")
    s = lax.axis_index("s")
    wid = s * NC + c
    base = wid * PW

    pltpu.sync_copy(ids_hbm.at[wid], idx_v)
    pltpu.sync_copy(ttf_hbm.at[wid], ttf_v)
    pltpu.sync_copy(pos_hbm.at[pl.ds(base, PW)], posv)
    pltpu.sync_copy(type_hbm, typ_v)
    pltpu.sync_copy(lnw_hbm, lnw_v)
    pltpu.sync_copy(lnb_hbm, lnb_v)

    def mkdiff(d, carry):
        sl = pl.ds(d * 16, 16)
        typ_v[1, sl] = typ_v[1, sl] - typ_v[0, sl]
        return carry

    lax.fori_loop(0, NV, mkdiff, 0)

    zero = jnp.zeros((16,), jnp.float32)
    lane = lax.iota(jnp.int32, 16)

    dnums = lax.GatherDimensionNumbers(
        offset_dims=(), collapsed_slice_dims=(0,), start_index_map=(0,))

    def lanesum(v):
        for k in (8, 4, 2, 1):
            v = v + lax.gather(
                v, (lane ^ k)[:, None], dnums, slice_sizes=(1,),
                mode=lax.GatherScatterMode.PROMISE_IN_BOUNDS)
        return v

    for b in range(BATCH):
        pltpu.async_copy(word_hbm.at[idx_v.at[b]], rows, sem).wait()

        def row_body(r, carry, b=b):
            ttf = ttf_v[pl.ds(b * PW + r, 16)][0]

            def p1(d, acc):
                acc_s, acc_q = acc
                sl = pl.ds(d * 16, 16)
                v = (rows[r, sl] + posv[r, sl]
                     + typ_v[0, sl] + ttf * typ_v[1, sl])
                rows[r, sl] = v
                return acc_s + v, acc_q + v * v

            acc_s, acc_q = lax.fori_loop(0, NV, p1, (zero, zero))
            mean = lanesum(acc_s) * (1.0 / HIDDEN)
            var = lanesum(acc_q) * (1.0 / HIDDEN) - mean * mean
            x = var[0] + EPS
            i = lax.bitcast_convert_type(x, jnp.int32)
            y = lax.bitcast_convert_type(
                jnp.int32(0x5F3759DF) - (i >> 1), jnp.float32)
            for _ in range(3):
                y = y * (1.5 - 0.5 * x * y * y)

            def p2(d, carry2):
                sl = pl.ds(d * 16, 16)
                v = rows[r, sl]
                rows[r, sl] = ((v - mean) * y) * lnw_v[sl] + lnb_v[sl]
                return carry2

            lax.fori_loop(0, NV, p2, 0)
            return carry

        lax.fori_loop(0, PW, row_body, 0)
        pltpu.sync_copy(rows, out_hbm.at[b, pl.ds(base, PW)])


@jax.jit
def _emb_ln(word_emb, ids_r, ttf_r, pos_emb, type_emb, ln_weight, ln_bias):
    mesh = plsc.VectorSubcoreMesh(
        core_axis_name="c", subcore_axis_name="s",
        num_cores=NC, num_subcores=NS)
    return pl.kernel(
        _body,
        out_type=jax.ShapeDtypeStruct((BATCH, SEQ, HIDDEN), jnp.float32),
        mesh=mesh,
        scratch_types=[
            pltpu.VMEM((BATCH, PW), jnp.int32),
            pltpu.VMEM((BATCH * PW + 16,), jnp.float32),
            pltpu.VMEM((PW, HIDDEN), jnp.float32),
            pltpu.VMEM((PW, HIDDEN), jnp.float32),
            pltpu.VMEM((2, HIDDEN), jnp.float32),
            pltpu.VMEM((HIDDEN,), jnp.float32),
            pltpu.VMEM((HIDDEN,), jnp.float32),
            pltpu.SemaphoreType.DMA,
        ],
    )(word_emb, ids_r, ttf_r, pos_emb, type_emb, ln_weight, ln_bias)


def kernel(input_ids, token_type_ids, word_emb, pos_emb, type_emb,
           ln_weight, ln_bias):
    ids_r = input_ids.reshape(BATCH, NW, PW).transpose(1, 0, 2)
    ids_r = ids_r.astype(jnp.int32)
    ttf_r = token_type_ids.reshape(BATCH, NW, PW).transpose(1, 0, 2)
    ttf_r = ttf_r.astype(jnp.float32).reshape(NW, BATCH * PW)
    ttf_r = jnp.pad(ttf_r, ((0, 0), (0, 16)))
    return _emb_ln(word_emb, ids_r, ttf_r, pos_emb, type_emb,
                   ln_weight, ln_bias)


# speedup vs baseline: 1.1395x; 1.1395x over previous
import functools

import jax
import jax.numpy as jnp
from jax import lax
from jax.experimental import pallas as pl
from jax.experimental.pallas import tpu as pltpu
from jax.experimental.pallas import tpu_sc as plsc

VOCAB = 100000
HIDDEN = 768
MAX_POS = 2048
BATCH = 4
SEQ = 2048
EPS = 1e-12

NC = 2
NS = 16
NW = NC * NS
PW = SEQ // NW
NV = HIDDEN // 16


def _body(word_hbm, ids_hbm, ttf_hbm, pos_hbm, type_hbm, lnw_hbm, lnb_hbm,
          out_hbm, idx_v, ttf_v, posv, rows, typ_v, lnw_v, lnb_v, sem):
    c = lax.axis_index("c")
    s = lax.axis_index("s")
    wid = s * NC + c
    base = wid * PW

    pltpu.sync_copy(ids_hbm.at[wid], idx_v)
    pltpu.sync_copy(ttf_hbm.at[wid], ttf_v)
    pltpu.sync_copy(pos_hbm.at[pl.ds(base, PW)], posv)
    pltpu.sync_copy(type_hbm, typ_v)
    pltpu.sync_copy(lnw_hbm, lnw_v)
    pltpu.sync_copy(lnb_hbm, lnb_v)

    def mkdiff(d, carry):
        sl = pl.ds(d * 16, 16)
        typ_v[1, sl] = typ_v[1, sl] - typ_v[0, sl]
        return carry

    lax.fori_loop(0, NV, mkdiff, 0, unroll=8)

    def fold_row(r, carry):
        def fold_d(d, c2):
            sl = pl.ds(d * 16, 16)
            posv[r, sl] = posv[r, sl] + typ_v[0, sl]
            return c2
        return lax.fori_loop(0, NV, fold_d, carry, unroll=8)

    lax.fori_loop(0, PW, fold_row, 0)

    zero = jnp.zeros((16,), jnp.float32)
    lane = lax.iota(jnp.int32, 16)

    dnums = lax.GatherDimensionNumbers(
        offset_dims=(), collapsed_slice_dims=(0,), start_index_map=(0,))

    def lanesum(v):
        for k in (8, 4, 2, 1):
            v = v + lax.gather(
                v, (lane ^ k)[:, None], dnums, slice_sizes=(1,),
                mode=lax.GatherScatterMode.PROMISE_IN_BOUNDS)
        return v

    for b in range(BATCH):
        pltpu.async_copy(word_hbm.at[idx_v.at[b]], rows, sem).wait()

        def row_body(r, carry, b=b):
            ttf = ttf_v[pl.ds(b * PW + r, 16)][0]

            def p1(d, acc):
                acc_s, acc_q = acc
                sl = pl.ds(d * 16, 16)
                v = rows[r, sl] + posv[r, sl] + ttf * typ_v[1, sl]
                rows[r, sl] = v
                return acc_s + v, acc_q + v * v

            acc_s, acc_q = lax.fori_loop(0, NV, p1, (zero, zero), unroll=8)
            mean = lanesum(acc_s) * (1.0 / HIDDEN)
            var = lanesum(acc_q) * (1.0 / HIDDEN) - mean * mean
            x = var[0] + EPS
            i = lax.bitcast_convert_type(x, jnp.int32)
            y = lax.bitcast_convert_type(
                jnp.int32(0x5F3759DF) - (i >> 1), jnp.float32)
            for _ in range(3):
                y = y * (1.5 - 0.5 * x * y * y)

            def p2(d, carry2):
                sl = pl.ds(d * 16, 16)
                v = rows[r, sl]
                rows[r, sl] = ((v - mean) * y) * lnw_v[sl] + lnb_v[sl]
                return carry2

            lax.fori_loop(0, NV, p2, 0, unroll=8)
            return carry

        lax.fori_loop(0, PW, row_body, 0)
        pltpu.sync_copy(rows, out_hbm.at[b, pl.ds(base, PW)])


@jax.jit
def _emb_ln(word_emb, ids_r, ttf_r, pos_emb, type_emb, ln_weight, ln_bias):
    mesh = plsc.VectorSubcoreMesh(
        core_axis_name="c", subcore_axis_name="s",
        num_cores=NC, num_subcores=NS)
    return pl.kernel(
        _body,
        out_type=jax.ShapeDtypeStruct((BATCH, SEQ, HIDDEN), jnp.float32),
        mesh=mesh,
        scratch_types=[
            pltpu.VMEM((BATCH, PW), jnp.int32),
            pltpu.VMEM((BATCH * PW + 16,), jnp.float32),
            pltpu.VMEM((PW, HIDDEN), jnp.float32),
            pltpu.VMEM((PW, HIDDEN), jnp.float32),
            pltpu.VMEM((2, HIDDEN), jnp.float32),
            pltpu.VMEM((HIDDEN,), jnp.float32),
            pltpu.VMEM((HIDDEN,), jnp.float32),
            pltpu.SemaphoreType.DMA,
        ],
    )(word_emb, ids_r, ttf_r, pos_emb, type_emb, ln_weight, ln_bias)


def kernel(input_ids, token_type_ids, word_emb, pos_emb, type_emb,
           ln_weight, ln_bias):
    ids_r = input_ids.reshape(BATCH, NW, PW).transpose(1, 0, 2)
    ids_r = ids_r.astype(jnp.int32)
    ttf_r = token_type_ids.reshape(BATCH, NW, PW).transpose(1, 0, 2)
    ttf_r = ttf_r.astype(jnp.float32).reshape(NW, BATCH * PW)
    ttf_r = jnp.pad(ttf_r, ((0, 0), (0, 16)))
    return _emb_ln(word_emb, ids_r, ttf_r, pos_emb, type_emb,
                   ln_weight, ln_bias)


# speedup vs baseline: 1.5165x; 1.3308x over previous
import functools

import jax
import jax.numpy as jnp
from jax import lax
from jax.experimental import pallas as pl
from jax.experimental.pallas import tpu as pltpu
from jax.experimental.pallas import tpu_sc as plsc

VOCAB = 100000
HIDDEN = 768
MAX_POS = 2048
BATCH = 4
SEQ = 2048
EPS = 1e-12

NC = 2
NS = 16
NW = NC * NS
PW = SEQ // NW
NV = HIDDEN // 16


def _body(word_hbm, ids_hbm, ttf_hbm, pos_hbm, type_hbm, lnw_hbm, lnb_hbm,
          out_hbm, idx_v, ttf_v, posv, rows, typ_v, lnw_v, lnb_v, sem):
    c = lax.axis_index("c")
    s = lax.axis_index("s")
    wid = s * NC + c
    base = wid * PW

    pltpu.sync_copy(ids_hbm.at[wid], idx_v)
    pltpu.sync_copy(ttf_hbm.at[wid], ttf_v)
    pltpu.sync_copy(pos_hbm.at[pl.ds(base, PW)], posv)
    pltpu.sync_copy(type_hbm, typ_v)
    pltpu.sync_copy(lnw_hbm, lnw_v)
    pltpu.sync_copy(lnb_hbm, lnb_v)

    @plsc.parallel_loop(0, NV, unroll=8)
    def mkdiff(d):
        sl = pl.ds(d * 16, 16)
        typ_v[1, sl] = typ_v[1, sl] - typ_v[0, sl]

    @plsc.parallel_loop(0, PW)
    def fold_row(r):
        @plsc.parallel_loop(0, NV, unroll=8)
        def fold_d(d):
            sl = pl.ds(d * 16, 16)
            posv[r, sl] = posv[r, sl] + typ_v[0, sl]

    zero = jnp.zeros((16,), jnp.float32)
    lane = lax.iota(jnp.int32, 16)

    dnums = lax.GatherDimensionNumbers(
        offset_dims=(), collapsed_slice_dims=(0,), start_index_map=(0,))

    def lanesum(v):
        for k in (8, 4, 2, 1):
            v = v + lax.gather(
                v, (lane ^ k)[:, None], dnums, slice_sizes=(1,),
                mode=lax.GatherScatterMode.PROMISE_IN_BOUNDS)
        return v

    for b in range(BATCH):
        pltpu.async_copy(word_hbm.at[idx_v.at[b]], rows, sem).wait()

        @plsc.parallel_loop(0, PW)
        def row_body(r, b=b):
            ttf = ttf_v[pl.ds(b * PW + r, 16)][0]

            def p1(i, acc):
                acc = list(acc)
                for j in range(4):
                    sl = pl.ds((i + j) * 16, 16)
                    v = rows[r, sl] + posv[r, sl] + ttf * typ_v[1, sl]
                    rows[r, sl] = v
                    acc[2 * j] = acc[2 * j] + v
                    acc[2 * j + 1] = acc[2 * j + 1] + v * v
                return tuple(acc)

            acc = plsc.parallel_loop(
                0, NV, step=4, unroll=2, carry=(zero,) * 8)(p1)
            acc_s = (acc[0] + acc[2]) + (acc[4] + acc[6])
            acc_q = (acc[1] + acc[3]) + (acc[5] + acc[7])
            mean = lanesum(acc_s) * (1.0 / HIDDEN)
            var = lanesum(acc_q) * (1.0 / HIDDEN) - mean * mean
            x = var[0] + EPS
            i = lax.bitcast_convert_type(x, jnp.int32)
            y = lax.bitcast_convert_type(
                jnp.int32(0x5F3759DF) - (i >> 1), jnp.float32)
            for _ in range(3):
                y = y * (1.5 - 0.5 * x * y * y)

            @plsc.parallel_loop(0, NV, unroll=8)
            def p2(d):
                sl = pl.ds(d * 16, 16)
                v = rows[r, sl]
                rows[r, sl] = ((v - mean) * y) * lnw_v[sl] + lnb_v[sl]
        pltpu.sync_copy(rows, out_hbm.at[b, pl.ds(base, PW)])


@jax.jit
def _emb_ln(word_emb, ids_r, ttf_r, pos_emb, type_emb, ln_weight, ln_bias):
    mesh = plsc.VectorSubcoreMesh(
        core_axis_name="c", subcore_axis_name="s",
        num_cores=NC, num_subcores=NS)
    return pl.kernel(
        _body,
        out_type=jax.ShapeDtypeStruct((BATCH, SEQ, HIDDEN), jnp.float32),
        mesh=mesh,
        scratch_types=[
            pltpu.VMEM((BATCH, PW), jnp.int32),
            pltpu.VMEM((BATCH * PW + 16,), jnp.float32),
            pltpu.VMEM((PW, HIDDEN), jnp.float32),
            pltpu.VMEM((PW, HIDDEN), jnp.float32),
            pltpu.VMEM((2, HIDDEN), jnp.float32),
            pltpu.VMEM((HIDDEN,), jnp.float32),
            pltpu.VMEM((HIDDEN,), jnp.float32),
            pltpu.SemaphoreType.DMA,
        ],
    )(word_emb, ids_r, ttf_r, pos_emb, type_emb, ln_weight, ln_bias)


def kernel(input_ids, token_type_ids, word_emb, pos_emb, type_emb,
           ln_weight, ln_bias):
    ids_r = input_ids.reshape(BATCH, NW, PW).transpose(1, 0, 2)
    ids_r = ids_r.astype(jnp.int32)
    ttf_r = token_type_ids.reshape(BATCH, NW, PW).transpose(1, 0, 2)
    ttf_r = ttf_r.astype(jnp.float32).reshape(NW, BATCH * PW)
    ttf_r = jnp.pad(ttf_r, ((0, 0), (0, 16)))
    return _emb_ln(word_emb, ids_r, ttf_r, pos_emb, type_emb,
                   ln_weight, ln_bias)


# speedup vs baseline: 1.6132x; 1.0638x over previous
import jax
import jax.numpy as jnp
from jax import lax
from jax.experimental import pallas as pl
from jax.experimental.pallas import tpu as pltpu
from jax.experimental.pallas import tpu_sc as plsc

VOCAB = 100000
HIDDEN = 768
MAX_POS = 2048
BATCH = 4
SEQ = 2048
EPS = 1e-12

NC = 2
NS = 16
NW = NC * NS
PW = SEQ // NW
NV = HIDDEN // 16
CR = 32
NCH = BATCH * PW // CR


def _body(word_hbm, ids_hbm, ttf_hbm, pos_hbm, type_hbm, lnw_hbm, lnb_hbm,
          out_hbm, idx_v, ttf_v, posv, rows3, typ_v, lnw_v, lnb_v,
          gsem, wsem):
    c_ax = lax.axis_index("c")
    s_ax = lax.axis_index("s")
    wid = s_ax * NC + c_ax
    base = wid * PW

    pltpu.sync_copy(ids_hbm.at[wid], idx_v)
    pltpu.sync_copy(ttf_hbm.at[wid], ttf_v)
    pltpu.sync_copy(pos_hbm.at[pl.ds(base, PW)], posv)
    pltpu.sync_copy(type_hbm, typ_v)
    pltpu.sync_copy(lnw_hbm, lnw_v)
    pltpu.sync_copy(lnb_hbm, lnb_v)

    def gather(c):
        return pltpu.async_copy(
            word_hbm.at[idx_v.at[c]], rows3.at[c % 3], gsem.at[c % 3])

    def writeout(c):
        b, half = divmod(c, 2)
        return pltpu.async_copy(
            rows3.at[c % 3],
            out_hbm.at[b, pl.ds(base + half * CR, CR)], wsem.at[c % 3])

    g = {0: gather(0), 1: gather(1)}
    w = {}

    @plsc.parallel_loop(0, NV, unroll=8)
    def mkdiff(d):
        sl = pl.ds(d * 16, 16)
        typ_v[1, sl] = typ_v[1, sl] - typ_v[0, sl]

    @plsc.parallel_loop(0, PW)
    def fold_row(r):
        @plsc.parallel_loop(0, NV, unroll=8)
        def fold_d(d):
            sl = pl.ds(d * 16, 16)
            posv[r, sl] = posv[r, sl] + typ_v[0, sl]

    zero = jnp.zeros((16,), jnp.float32)
    lane = lax.iota(jnp.int32, 16)

    dnums = lax.GatherDimensionNumbers(
        offset_dims=(), collapsed_slice_dims=(0,), start_index_map=(0,))

    def lanesum(v):
        for k in (8, 4, 2, 1):
            v = v + lax.gather(
                v, (lane ^ k)[:, None], dnums, slice_sizes=(1,),
                mode=lax.GatherScatterMode.PROMISE_IN_BOUNDS)
        return v

    for c in range(NCH):
        k = c % 3
        g[c].wait()

        @plsc.parallel_loop(0, CR)
        def row_body(r, c=c, k=k):
            poff = (c % 2) * CR
            ttf = ttf_v[pl.ds(c * CR + r, 16)][0]

            def p1(i, acc):
                acc = list(acc)
                for j in range(4):
                    sl = pl.ds((i + j) * 16, 16)
                    v = (rows3[k, r, sl] + posv[poff + r, sl]
                         + ttf * typ_v[1, sl])
                    rows3[k, r, sl] = v
                    acc[2 * j] = acc[2 * j] + v
                    acc[2 * j + 1] = acc[2 * j + 1] + v * v
                return tuple(acc)

            acc = plsc.parallel_loop(
                0, NV, step=4, unroll=2, carry=(zero,) * 8)(p1)
            acc_s = (acc[0] + acc[2]) + (acc[4] + acc[6])
            acc_q = (acc[1] + acc[3]) + (acc[5] + acc[7])
            mean = lanesum(acc_s) * (1.0 / HIDDEN)
            var = lanesum(acc_q) * (1.0 / HIDDEN) - mean * mean
            x = var[0] + EPS
            i = lax.bitcast_convert_type(x, jnp.int32)
            y = lax.bitcast_convert_type(
                jnp.int32(0x5F3759DF) - (i >> 1), jnp.float32)
            for _ in range(3):
                y = y * (1.5 - 0.5 * x * y * y)

            @plsc.parallel_loop(0, NV, unroll=8)
            def p2(d):
                sl = pl.ds(d * 16, 16)
                v = rows3[k, r, sl]
                rows3[k, r, sl] = ((v - mean) * y) * lnw_v[sl] + lnb_v[sl]

        if c + 2 < NCH:
            if c - 1 >= 0:
                w[c - 1].wait()
            g[c + 2] = gather(c + 2)
        w[c] = writeout(c)

    w[NCH - 2].wait()
    w[NCH - 1].wait()


@jax.jit
def _emb_ln(word_emb, ids_r, ttf_r, pos_emb, type_emb, ln_weight, ln_bias):
    mesh = plsc.VectorSubcoreMesh(
        core_axis_name="c", subcore_axis_name="s",
        num_cores=NC, num_subcores=NS)
    return pl.kernel(
        _body,
        out_type=jax.ShapeDtypeStruct((BATCH, SEQ, HIDDEN), jnp.float32),
        mesh=mesh,
        scratch_types=[
            pltpu.VMEM((NCH, CR), jnp.int32),
            pltpu.VMEM((BATCH * PW + 16,), jnp.float32),
            pltpu.VMEM((PW, HIDDEN), jnp.float32),
            pltpu.VMEM((3, CR, HIDDEN), jnp.float32),
            pltpu.VMEM((2, HIDDEN), jnp.float32),
            pltpu.VMEM((HIDDEN,), jnp.float32),
            pltpu.VMEM((HIDDEN,), jnp.float32),
            pltpu.SemaphoreType.DMA((3,)),
            pltpu.SemaphoreType.DMA((3,)),
        ],
    )(word_emb, ids_r, ttf_r, pos_emb, type_emb, ln_weight, ln_bias)


def kernel(input_ids, token_type_ids, word_emb, pos_emb, type_emb,
           ln_weight, ln_bias):
    ids_r = input_ids.reshape(BATCH, NW, PW).transpose(1, 0, 2)
    ids_r = ids_r.astype(jnp.int32).reshape(NW, NCH, CR)
    ttf_r = token_type_ids.reshape(BATCH, NW, PW).transpose(1, 0, 2)
    ttf_r = ttf_r.astype(jnp.float32).reshape(NW, BATCH * PW)
    ttf_r = jnp.pad(ttf_r, ((0, 0), (0, 16)))
    return _emb_ln(word_emb, ids_r, ttf_r, pos_emb, type_emb,
                   ln_weight, ln_bias)


# speedup vs baseline: 4.1008x; 2.5421x over previous
import jax
import jax.numpy as jnp
from jax import lax
from jax.experimental import pallas as pl
from jax.experimental.pallas import tpu as pltpu
from jax.experimental.pallas import tpu_sc as plsc

VOCAB = 100000
HIDDEN = 768
MAX_POS = 2048
BATCH = 4
SEQ = 2048
EPS = 1e-12

NC = 2
NS = 16
NW = NC * NS
TOK = BATCH * SEQ
TW = TOK // NW
CR = 64
NCH = TW // CR
NBUF = 2


def _sc_body(word_hbm, ids_hbm, gath_hbm, idx_v, rows, gsem, wsem):
    c_ax = lax.axis_index("c")
    s_ax = lax.axis_index("s")
    wid = s_ax * NC + c_ax
    base = wid * TW

    pltpu.sync_copy(ids_hbm.at[wid], idx_v)

    def gather(c):
        return pltpu.async_copy(
            word_hbm.at[idx_v.at[c]], rows.at[c % NBUF], gsem.at[c % NBUF])

    def writeout(c):
        return pltpu.async_copy(
            rows.at[c % NBUF],
            gath_hbm.at[pl.ds(base + c * CR, CR)], wsem.at[c % NBUF])

    g = {0: gather(0), 1: gather(1)}
    w = {}
    for c in range(NCH):
        g[c].wait()
        w[c] = writeout(c)
        if c + NBUF < NCH:
            w[c].wait()
            g[c + NBUF] = gather(c + NBUF)
    for c in range(NCH - NBUF, NCH):
        w[c].wait()


@jax.jit
def _sc_gather(word_emb, ids_r):
    mesh = plsc.VectorSubcoreMesh(
        core_axis_name="c", subcore_axis_name="s",
        num_cores=NC, num_subcores=NS)
    return pl.kernel(
        _sc_body,
        out_type=jax.ShapeDtypeStruct((TOK, HIDDEN), jnp.float32),
        mesh=mesh,
        scratch_types=[
            pltpu.VMEM((NCH, CR), jnp.int32),
            pltpu.VMEM((NBUF, CR, HIDDEN), jnp.float32),
            pltpu.SemaphoreType.DMA((NBUF,)),
            pltpu.SemaphoreType.DMA((NBUF,)),
        ],
    )(word_emb, ids_r)


def _tc_body(gath_ref, pos_ref, ttf_ref, typ_ref, lnw_ref, lnb_ref, out_ref):
    x = gath_ref[...]
    t0 = typ_ref[0, :]
    tdiff = typ_ref[1, :] - t0
    ttf = ttf_ref[0, 0, :]
    x = x + pos_ref[...] + t0[None, :] + ttf[:, None] * tdiff[None, :]
    mean = jnp.mean(x, axis=-1, keepdims=True)
    xc = x - mean
    var = jnp.mean(xc * xc, axis=-1, keepdims=True)
    nrm = xc * lax.rsqrt(var + EPS)
    out_ref[...] = nrm * lnw_ref[0, :][None, :] + lnb_ref[0, :][None, :]


@jax.jit
def _tc_add_ln(gathered, pos_emb, ttf, type_emb, lnw2, lnb2):
    return pl.pallas_call(
        _tc_body,
        grid=(BATCH,),
        in_specs=[
            pl.BlockSpec((SEQ, HIDDEN), lambda i: (i, 0)),
            pl.BlockSpec((SEQ, HIDDEN), lambda i: (0, 0)),
            pl.BlockSpec((1, 1, SEQ), lambda i: (i, 0, 0)),
            pl.BlockSpec((2, HIDDEN), lambda i: (0, 0)),
            pl.BlockSpec((1, HIDDEN), lambda i: (0, 0)),
            pl.BlockSpec((1, HIDDEN), lambda i: (0, 0)),
        ],
        out_specs=pl.BlockSpec((SEQ, HIDDEN), lambda i: (i, 0)),
        out_shape=jax.ShapeDtypeStruct((TOK, HIDDEN), jnp.float32),
    )(gathered, pos_emb, ttf, type_emb, lnw2, lnb2)


def kernel(input_ids, token_type_ids, word_emb, pos_emb, type_emb,
           ln_weight, ln_bias):
    ids_r = input_ids.astype(jnp.int32).reshape(NW, NCH, CR)
    gathered = _sc_gather(word_emb, ids_r)
    ttf = token_type_ids.astype(jnp.float32).reshape(BATCH, 1, SEQ)
    lnw2 = ln_weight.reshape(1, HIDDEN)
    lnb2 = ln_bias.reshape(1, HIDDEN)
    out = _tc_add_ln(gathered, pos_emb, ttf, type_emb, lnw2, lnb2)
    return out.reshape(BATCH, SEQ, HIDDEN)
